# baseline (speedup 1.0000x reference)
import jax
import jax.numpy as jnp
from jax.experimental import pallas as pl
from jax.experimental.pallas import tpu as pltpu

_OC = 64
_HID = 512
_L = 32
_C = 20


def _fused_kernel(x_ref, wc_ref, bc_ref, wl_ref, bl_ref, o_ref,
                  wband, bcs, wls, bls):
    j = pl.program_id(0)

    @pl.when(j == 0)
    def _prep():
        w1t = jnp.transpose(wc_ref[:, 1, :]).astype(jnp.bfloat16)
        w0t = jnp.transpose(wc_ref[:, 0, :]).astype(jnp.bfloat16)
        nr, nl = (_L + 1) * _OC, _C * _L
        w1e = jnp.repeat(w1t, _L, axis=1)
        w0e = jnp.repeat(w0t, _L, axis=1)
        vals1 = jnp.concatenate([w1e] * (_L + 1), axis=0)
        vals0 = jnp.concatenate([w0e] * (_L + 1), axis=0)
        t_of_lane = jax.lax.broadcasted_iota(jnp.int32, (nr, nl), 1) % _L
        p_of_row = jax.lax.broadcasted_iota(jnp.int32, (nr, nl), 0) // _OC
        zero = jnp.zeros((), jnp.bfloat16)
        wband[...] = (jnp.where(t_of_lane == p_of_row, vals1, zero)
                      + jnp.where(t_of_lane + 1 == p_of_row, vals0, zero))
        wls[...] = (wl_ref[...].astype(jnp.bfloat16)
                    .reshape(_HID, _OC, _L)
                    .transpose(0, 2, 1).reshape(_HID, _L * _OC))
        bcol = jnp.transpose(bc_ref[...])
        bcs[...] = jnp.concatenate([bcol] * _L, axis=0)
        bls[...] = (jnp.dot(wls[...].astype(jnp.float32), bcs[...],
                            preferred_element_type=jnp.float32)
                    + jnp.transpose(bl_ref[...]))

    xb = x_ref[...].astype(jnp.bfloat16)
    convT = jnp.dot(wband[...], xb,
                    preferred_element_type=jnp.float32)
    convb = convT.astype(jnp.bfloat16)
    pooledT = jnp.maximum(convb[:_L * _OC], convb[_OC:])
    outT = (jnp.dot(wls[...], pooledT,
                    preferred_element_type=jnp.float32)
            + bls[...])
    o_ref[...] = jnp.transpose(outT).astype(o_ref.dtype)


def kernel(protein_ft, w_conv, b_conv, w_lin, b_lin):
    B, L, C = protein_ft.shape
    assert (L, C) == (_L, _C), (L, C)
    f32 = jnp.float32

    xt = protein_ft.transpose(2, 1, 0).reshape(C * L, B).astype(f32)
    TB = 512 if B >= 512 else -(-B // 128) * 128
    B_pad = -(-B // TB) * TB
    if B_pad != B:
        xt = jnp.pad(xt, ((0, 0), (0, B_pad - B)))
    nbt = B_pad // TB

    wcf = w_conv.transpose(1, 2, 0).astype(f32)
    bcf = b_conv.astype(f32)[None, :]
    wlf = w_lin.astype(f32)
    blf = b_lin.astype(f32)[None, :]

    out = pl.pallas_call(
        _fused_kernel,
        out_shape=jax.ShapeDtypeStruct((B_pad, _HID), f32),
        grid=(nbt,),
        in_specs=[
            pl.BlockSpec((C * L, TB), lambda j: (0, j)),
            pl.BlockSpec(wcf.shape, lambda j: (0, 0, 0)),
            pl.BlockSpec((1, _OC), lambda j: (0, 0)),
            pl.BlockSpec((_HID, L * _OC), lambda j: (0, 0)),
            pl.BlockSpec((1, _HID), lambda j: (0, 0)),
        ],
        out_specs=pl.BlockSpec((TB, _HID), lambda j: (j, 0)),
        scratch_shapes=[
            pltpu.VMEM(((_L + 1) * _OC, _C * _L), jnp.bfloat16),
            pltpu.VMEM((_L * _OC, 1), f32),
            pltpu.VMEM((_HID, _L * _OC), jnp.bfloat16),
            pltpu.VMEM((_HID, 1), f32),
        ],
        compiler_params=pltpu.CompilerParams(
            dimension_semantics=("arbitrary",),
            vmem_limit_bytes=64 << 20),
    )(xt, wcf, bcf, wlf, blf)
    return out[:B]


# speedup vs baseline: 1.0536x; 1.0536x over previous
import jax
import jax.numpy as jnp
from jax.experimental import pallas as pl
from jax.experimental.pallas import tpu as pltpu

_OC = 64
_HID = 512
_L = 32
_C = 20


def _fused_kernel(x_ref, wc_ref, bc_ref, wl_ref, bl_ref, o_ref,
                  wband, bcs, wls, bls):
    j = pl.program_id(0)

    @pl.when(j == 0)
    def _prep():
        w1t = jnp.transpose(wc_ref[:, 1, :]).astype(jnp.bfloat16)
        w0t = jnp.transpose(wc_ref[:, 0, :]).astype(jnp.bfloat16)
        nr, nl = (_L + 1) * _OC, _C * _L
        w1e = jnp.repeat(w1t, _L, axis=1)
        w0e = jnp.repeat(w0t, _L, axis=1)
        vals1 = jnp.concatenate([w1e] * (_L + 1), axis=0)
        vals0 = jnp.concatenate([w0e] * (_L + 1), axis=0)
        t_of_lane = jax.lax.broadcasted_iota(jnp.int32, (nr, nl), 1) % _L
        p_of_row = jax.lax.broadcasted_iota(jnp.int32, (nr, nl), 0) // _OC
        zero = jnp.zeros((), jnp.bfloat16)
        wband[...] = (jnp.where(t_of_lane == p_of_row, vals1, zero)
                      + jnp.where(t_of_lane + 1 == p_of_row, vals0, zero))
        wls[...] = (wl_ref[...].astype(jnp.bfloat16)
                    .reshape(_HID, _OC, _L)
                    .transpose(0, 2, 1).reshape(_HID, _L * _OC))
        bcol = jnp.transpose(bc_ref[...])
        bcs[...] = jnp.concatenate([bcol] * _L, axis=0)
        bls[...] = (jnp.dot(wls[...].astype(jnp.float32), bcs[...],
                            preferred_element_type=jnp.float32)
                    + jnp.transpose(bl_ref[...]))

    xb = x_ref[...].astype(jnp.bfloat16)
    convT = jnp.dot(wband[...], xb,
                    preferred_element_type=jnp.float32)
    convb = convT.astype(jnp.bfloat16)
    pooledT = jnp.maximum(convb[:_L * _OC], convb[_OC:])
    outT = (jnp.dot(wls[...], pooledT,
                    preferred_element_type=jnp.float32)
            + bls[...])
    o_ref[...] = jnp.transpose(outT).astype(o_ref.dtype)


def kernel(protein_ft, w_conv, b_conv, w_lin, b_lin):
    B, L, C = protein_ft.shape
    assert (L, C) == (_L, _C), (L, C)
    f32 = jnp.float32

    xt = protein_ft.transpose(2, 1, 0).reshape(C * L, B).astype(f32)
    TB = 1024 if B >= 1024 else -(-B // 128) * 128
    B_pad = -(-B // TB) * TB
    if B_pad != B:
        xt = jnp.pad(xt, ((0, 0), (0, B_pad - B)))
    nbt = B_pad // TB

    wcf = w_conv.transpose(1, 2, 0).astype(f32)
    bcf = b_conv.astype(f32)[None, :]
    wlf = w_lin.astype(f32)
    blf = b_lin.astype(f32)[None, :]

    out = pl.pallas_call(
        _fused_kernel,
        out_shape=jax.ShapeDtypeStruct((B_pad, _HID), f32),
        grid=(nbt,),
        in_specs=[
            pl.BlockSpec((C * L, TB), lambda j: (0, j)),
            pl.BlockSpec(wcf.shape, lambda j: (0, 0, 0)),
            pl.BlockSpec((1, _OC), lambda j: (0, 0)),
            pl.BlockSpec((_HID, L * _OC), lambda j: (0, 0)),
            pl.BlockSpec((1, _HID), lambda j: (0, 0)),
        ],
        out_specs=pl.BlockSpec((TB, _HID), lambda j: (j, 0)),
        scratch_shapes=[
            pltpu.VMEM(((_L + 1) * _OC, _C * _L), jnp.bfloat16),
            pltpu.VMEM((_L * _OC, 1), f32),
            pltpu.VMEM((_HID, _L * _OC), jnp.bfloat16),
            pltpu.VMEM((_HID, 1), f32),
        ],
        compiler_params=pltpu.CompilerParams(
            dimension_semantics=("arbitrary",),
            vmem_limit_bytes=64 << 20),
    )(xt, wcf, bcf, wlf, blf)
    return out[:B]
